# baseline (speedup 1.0000x reference)
import functools

import jax
import jax.numpy as jnp
from jax import lax
from jax.experimental import pallas as pl
from jax.experimental.pallas import tpu as pltpu
from jax.experimental.pallas import tpu_sc as plsc

VOCAB = 1000000
SEQ_LEN = 200
EMBED_DIM = 32
BATCH = 4096

NC = 2
NS = 16
NW = NC * NS
L = 16

BTILE = 128
NJ = BATCH // BTILE
UNITS = SEQ_LEN * NJ
UNITS_PER_W = UNITS // NW
IDX_PER_W = UNITS_PER_W * BTILE
TPITCH = BTILE + 1


def _make_kernel():
    mesh = plsc.VectorSubcoreMesh(core_axis_name="c", subcore_axis_name="s")

    @functools.partial(
        pl.kernel,
        mesh=mesh,
        out_type=jax.ShapeDtypeStruct((SEQ_LEN * 4, NJ, 8, BTILE),
                                      jnp.float32),
        compiler_params=pltpu.CompilerParams(use_tc_tiling_on_sc=False,
                                             needs_layout_passes=False),
        scratch_types=[
            pltpu.VMEM((IDX_PER_W,), jnp.int32),
            pltpu.VMEM((2, BTILE, EMBED_DIM), jnp.float32),
            pltpu.VMEM((2, EMBED_DIM, TPITCH), jnp.float32),
            pltpu.VMEM((SEQ_LEN, EMBED_DIM), jnp.float32),
            pltpu.SemaphoreType.DMA((2,)),
            pltpu.SemaphoreType.DMA((2,)),
        ],
    )
    def k(idx_hbm, table_hbm, pos_hbm, out_hbm, idx_v, rows_v, trans_v,
          pos_v, gsem, osem):
        wid = lax.axis_index("s") * NC + lax.axis_index("c")
        g0 = wid * UNITS_PER_W

        pltpu.sync_copy(
            idx_hbm.at[pl.ds(pl.multiple_of(wid * IDX_PER_W, IDX_PER_W),
                             IDX_PER_W)],
            idx_v)
        pltpu.sync_copy(pos_hbm, pos_v)

        lane = lax.broadcasted_iota(jnp.int32, (L,), 0)

        def fire(uu, p):
            pltpu.async_copy(
                table_hbm.at[idx_v.at[pl.ds(uu * BTILE, BTILE)]],
                rows_v.at[p],
                gsem.at[p],
            )

        def drain_gather(p):
            pltpu.make_async_copy(
                table_hbm.at[pl.ds(0, BTILE)], rows_v.at[p], gsem.at[p]
            ).wait()

        def transform(uu, p):
            s = (g0 + uu) // NJ
            plo = pos_v[s, pl.ds(0, L)]
            phi = pos_v[s, pl.ds(L, L)]

            def rbody(rr, carry):
                for q in range(4):
                    r = rr * 4 + q
                    rcol = jnp.full((L,), 0, jnp.int32) + r
                    lo = rows_v[p, r, pl.ds(0, L)] + plo
                    hi = rows_v[p, r, pl.ds(L, L)] + phi
                    plsc.store_scatter(trans_v.at[p], [lane, rcol], lo)
                    plsc.store_scatter(trans_v.at[p], [lane + L, rcol], hi)
                return carry

            lax.fori_loop(0, BTILE // 4, rbody, 0)

        def fire_stores(uu, p):
            g = g0 + uu
            s = g // NJ
            j = g % NJ
            for i in range(4):
                pltpu.async_copy(
                    trans_v.at[p].at[pl.ds(i * 8, 8), pl.ds(0, BTILE)],
                    out_hbm.at[s * 4 + i, j],
                    osem.at[p],
                )

        def drain_stores(p):
            for i in range(4):
                pltpu.make_async_copy(
                    trans_v.at[p].at[pl.ds(i * 8, 8), pl.ds(0, BTILE)],
                    out_hbm.at[0, 0],
                    osem.at[p],
                ).wait()

        fire(0, 0)

        def pair(h, carry):
            uu = h * 2
            @pl.when(uu + 1 < UNITS_PER_W)
            def _():
                fire(uu + 1, 1)
            drain_gather(0)
            @pl.when(uu >= 2)
            def _():
                drain_stores(0)
            transform(uu, 0)
            fire_stores(uu, 0)
            @pl.when(uu + 2 < UNITS_PER_W)
            def _():
                fire(uu + 2, 0)
            drain_gather(1)
            @pl.when(uu >= 2)
            def _():
                drain_stores(1)
            transform(uu + 1, 1)
            fire_stores(uu + 1, 1)
            return carry

        lax.fori_loop(0, UNITS_PER_W // 2, pair, 0)
        drain_stores(0)
        drain_stores(1)

    return k


_sc_kernel = _make_kernel()

TC_C = 512
TC_GRID = (VOCAB + TC_C - 1) // TC_C


def _tbody(x_ref, o_ref):
    w = x_ref[...].T
    wr = w.reshape(TC_C // 4, 4, EMBED_DIM)
    o_ref[...] = jnp.concatenate([wr[:, k, :] for k in range(4)], axis=1)


def _transpose_table(tT):
    return pl.pallas_call(
        _tbody,
        grid=(TC_GRID,),
        in_specs=[pl.BlockSpec((EMBED_DIM, TC_C), lambda i: (0, i))],
        out_specs=pl.BlockSpec((TC_C // 4, 128), lambda i: (i, 0)),
        out_shape=jax.ShapeDtypeStruct((VOCAB // 4, 128), jnp.float32),
    )(tT)


@jax.jit
def kernel(inputs, token_table, pos_table):
    idx = inputs.astype(jnp.int32).T.reshape(-1)
    tbl = _transpose_table(token_table.T).reshape(VOCAB, EMBED_DIM)
    out4 = _sc_kernel(idx, tbl, pos_table)
    o5 = out4.reshape(SEQ_LEN, 4, NJ, 8, BTILE)
    return o5.transpose(2, 4, 0, 1, 3).reshape(BATCH, SEQ_LEN, EMBED_DIM)


# speedup vs baseline: 2.0374x; 2.0374x over previous
import functools

import jax
import jax.numpy as jnp
from jax import lax
from jax.experimental import pallas as pl
from jax.experimental.pallas import tpu as pltpu
from jax.experimental.pallas import tpu_sc as plsc

VOCAB = 1000000
SEQ_LEN = 200
EMBED_DIM = 32
BATCH = 4096

NC = 2
NS = 16
NW = NC * NS
L = 16

BTILE = 128
NJ = BATCH // BTILE
UNITS = SEQ_LEN * NJ
UNITS_PER_W = UNITS // NW
IDX_PER_W = UNITS_PER_W * BTILE
TPITCH = BTILE + 1


def _make_kernel():
    mesh = plsc.VectorSubcoreMesh(core_axis_name="c", subcore_axis_name="s")

    @functools.partial(
        pl.kernel,
        mesh=mesh,
        out_type=jax.ShapeDtypeStruct((SEQ_LEN * 4, NJ, 8, BTILE),
                                      jnp.float32),
        compiler_params=pltpu.CompilerParams(use_tc_tiling_on_sc=False,
                                             needs_layout_passes=False),
        scratch_types=[
            pltpu.VMEM((IDX_PER_W,), jnp.int32),
            pltpu.VMEM((2, BTILE, EMBED_DIM), jnp.float32),
            pltpu.VMEM((2, EMBED_DIM, TPITCH), jnp.float32),
            pltpu.VMEM((SEQ_LEN, EMBED_DIM), jnp.float32),
            pltpu.SemaphoreType.DMA((2,)),
            pltpu.SemaphoreType.DMA((2,)),
        ],
    )
    def k(idx_hbm, table_hbm, pos_hbm, out_hbm, idx_v, rows_v, trans_v,
          pos_v, gsem, osem):
        wid = lax.axis_index("s") * NC + lax.axis_index("c")
        g0 = wid * UNITS_PER_W

        pltpu.sync_copy(
            idx_hbm.at[pl.ds(pl.multiple_of(wid * IDX_PER_W, IDX_PER_W),
                             IDX_PER_W)],
            idx_v)
        pltpu.sync_copy(pos_hbm, pos_v)

        lane = lax.broadcasted_iota(jnp.int32, (L,), 0)

        def fire(uu, p):
            pltpu.async_copy(
                table_hbm.at[idx_v.at[pl.ds(uu * BTILE, BTILE)]],
                rows_v.at[p],
                gsem.at[p],
            )

        def drain_gather(p):
            pltpu.make_async_copy(
                table_hbm.at[pl.ds(0, BTILE)], rows_v.at[p], gsem.at[p]
            ).wait()

        def transform(uu, p):
            s = (g0 + uu) // NJ
            plo = pos_v[s, pl.ds(0, L)]
            phi = pos_v[s, pl.ds(L, L)]

            def rbody(rr, carry):
                for q in range(4):
                    r = rr * 4 + q
                    rcol = jnp.full((L,), 0, jnp.int32) + r
                    lo = rows_v[p, r, pl.ds(0, L)] + plo
                    hi = rows_v[p, r, pl.ds(L, L)] + phi
                    plsc.store_scatter(trans_v.at[p], [lane, rcol], lo)
                    plsc.store_scatter(trans_v.at[p], [lane + L, rcol], hi)
                return carry

            lax.fori_loop(0, BTILE // 4, rbody, 0)

        def fire_stores(uu, p):
            g = g0 + uu
            s = g // NJ
            j = g % NJ
            for i in range(4):
                pltpu.async_copy(
                    trans_v.at[p].at[pl.ds(i * 8, 8), pl.ds(0, BTILE)],
                    out_hbm.at[s * 4 + i, j],
                    osem.at[p],
                )

        def drain_stores(p):
            for i in range(4):
                pltpu.make_async_copy(
                    trans_v.at[p].at[pl.ds(i * 8, 8), pl.ds(0, BTILE)],
                    out_hbm.at[0, 0],
                    osem.at[p],
                ).wait()

        fire(0, 0)

        def pair(h, carry):
            uu = h * 2
            @pl.when(uu + 1 < UNITS_PER_W)
            def _():
                fire(uu + 1, 1)
            drain_gather(0)
            @pl.when(uu >= 2)
            def _():
                drain_stores(0)
            transform(uu, 0)
            fire_stores(uu, 0)
            @pl.when(uu + 2 < UNITS_PER_W)
            def _():
                fire(uu + 2, 0)
            drain_gather(1)
            @pl.when(uu >= 2)
            def _():
                drain_stores(1)
            transform(uu + 1, 1)
            fire_stores(uu + 1, 1)
            return carry

        lax.fori_loop(0, UNITS_PER_W // 2, pair, 0)
        drain_stores(0)
        drain_stores(1)

    return k


_sc_kernel = _make_kernel()


@jax.jit
def kernel(inputs, token_table, pos_table):
    idx = inputs.astype(jnp.int32).T.reshape(-1)
    out4 = _sc_kernel(idx, token_table, pos_table)
    o5 = out4.reshape(SEQ_LEN, 4, NJ, 8, BTILE)
    return o5.transpose(2, 4, 0, 1, 3).reshape(BATCH, SEQ_LEN, EMBED_DIM)


# speedup vs baseline: 2.1072x; 1.0342x over previous
import functools

import jax
import jax.numpy as jnp
from jax import lax
from jax.experimental import pallas as pl
from jax.experimental.pallas import tpu as pltpu
from jax.experimental.pallas import tpu_sc as plsc

VOCAB = 1000000
SEQ_LEN = 200
EMBED_DIM = 32
BATCH = 4096

NC = 2
NS = 16
NW = NC * NS
L = 16

BTILE = 128
NJ = BATCH // BTILE
UNITS = SEQ_LEN * NJ
UNITS_PER_W = UNITS // NW
IDX_PER_W = UNITS_PER_W * BTILE
TPITCH = BTILE + 1
NBUF = 4
AHEAD = 2


def _make_kernel():
    mesh = plsc.VectorSubcoreMesh(core_axis_name="c", subcore_axis_name="s")

    @functools.partial(
        pl.kernel,
        mesh=mesh,
        out_type=jax.ShapeDtypeStruct((SEQ_LEN * 4, NJ, 8, BTILE),
                                      jnp.float32),
        compiler_params=pltpu.CompilerParams(use_tc_tiling_on_sc=False,
                                             needs_layout_passes=False),
        scratch_types=[
            pltpu.VMEM((IDX_PER_W,), jnp.int32),
            pltpu.VMEM((NBUF, BTILE, EMBED_DIM), jnp.float32),
            pltpu.VMEM((NBUF, EMBED_DIM, TPITCH), jnp.float32),
            pltpu.VMEM((SEQ_LEN, EMBED_DIM), jnp.float32),
            pltpu.SemaphoreType.DMA((NBUF,)),
            pltpu.SemaphoreType.DMA((NBUF,)),
        ],
    )
    def k(idx_hbm, table_hbm, pos_hbm, out_hbm, idx_v, rows_v, trans_v,
          pos_v, gsem, osem):
        wid = lax.axis_index("s") * NC + lax.axis_index("c")
        g0 = wid * UNITS_PER_W

        pltpu.sync_copy(
            idx_hbm.at[pl.ds(pl.multiple_of(wid * IDX_PER_W, IDX_PER_W),
                             IDX_PER_W)],
            idx_v)
        pltpu.sync_copy(pos_hbm, pos_v)

        lane = lax.broadcasted_iota(jnp.int32, (L,), 0)

        def fire(uu, p):
            pltpu.async_copy(
                table_hbm.at[idx_v.at[pl.ds(uu * BTILE, BTILE)]],
                rows_v.at[p],
                gsem.at[p],
            )

        def drain_gather(p):
            pltpu.make_async_copy(
                table_hbm.at[pl.ds(0, BTILE)], rows_v.at[p], gsem.at[p]
            ).wait()

        def transform(uu, p):
            s = (g0 + uu) // NJ
            plo = pos_v[s, pl.ds(0, L)]
            phi = pos_v[s, pl.ds(L, L)]

            def rbody(rr, carry):
                for q in range(4):
                    r = rr * 4 + q
                    rcol = jnp.full((L,), 0, jnp.int32) + r
                    lo = rows_v[p, r, pl.ds(0, L)] + plo
                    hi = rows_v[p, r, pl.ds(L, L)] + phi
                    plsc.store_scatter(trans_v.at[p], [lane, rcol], lo)
                    plsc.store_scatter(trans_v.at[p], [lane + L, rcol], hi)
                return carry

            lax.fori_loop(0, BTILE // 4, rbody, 0)

        def fire_stores(uu, p):
            g = g0 + uu
            s = g // NJ
            j = g % NJ
            for i in range(4):
                pltpu.async_copy(
                    trans_v.at[p].at[pl.ds(i * 8, 8), pl.ds(0, BTILE)],
                    out_hbm.at[s * 4 + i, j],
                    osem.at[p],
                )

        def drain_stores(p):
            for i in range(4):
                pltpu.make_async_copy(
                    trans_v.at[p].at[pl.ds(i * 8, 8), pl.ds(0, BTILE)],
                    out_hbm.at[0, 0],
                    osem.at[p],
                ).wait()

        for a in range(AHEAD):
            fire(a, a)

        def quad(h, carry):
            uu = h * NBUF
            for q in range(NBUF):
                u = uu + q
                @pl.when(u + AHEAD < UNITS_PER_W)
                def _(u=u, q=q):
                    @pl.when(u + AHEAD >= NBUF)
                    def _():
                        drain_stores((q + AHEAD) % NBUF)
                    fire(u + AHEAD, (q + AHEAD) % NBUF)
                drain_gather(q)
                transform(u, q)
                fire_stores(u, q)
            return carry

        lax.fori_loop(0, UNITS_PER_W // NBUF, quad, 0)
        for q in range(NBUF):
            drain_stores(q)

    return k


_sc_kernel = _make_kernel()


@jax.jit
def kernel(inputs, token_table, pos_table):
    idx = inputs.astype(jnp.int32).T.reshape(-1)
    out4 = _sc_kernel(idx, token_table, pos_table)
    o5 = out4.reshape(SEQ_LEN, 4, NJ, 8, BTILE)
    return o5.transpose(2, 4, 0, 1, 3).reshape(BATCH, SEQ_LEN, EMBED_DIM)


# speedup vs baseline: 2.1082x; 1.0005x over previous
import functools

import jax
import jax.numpy as jnp
from jax import lax
from jax.experimental import pallas as pl
from jax.experimental.pallas import tpu as pltpu
from jax.experimental.pallas import tpu_sc as plsc

VOCAB = 1000000
SEQ_LEN = 200
EMBED_DIM = 32
BATCH = 4096

NC = 2
NS = 16
NW = NC * NS
L = 16

BTILE = 128
NJ = BATCH // BTILE
UNITS = SEQ_LEN * NJ
UNITS_PER_W = UNITS // NW
IDX_PER_W = UNITS_PER_W * BTILE
TPITCH = BTILE + 1
NBUF = 5
AHEAD = 2


def _make_kernel():
    mesh = plsc.VectorSubcoreMesh(core_axis_name="c", subcore_axis_name="s")

    @functools.partial(
        pl.kernel,
        mesh=mesh,
        out_type=jax.ShapeDtypeStruct((SEQ_LEN * 4, NJ, 8, BTILE),
                                      jnp.float32),
        compiler_params=pltpu.CompilerParams(use_tc_tiling_on_sc=False,
                                             needs_layout_passes=False),
        scratch_types=[
            pltpu.VMEM((IDX_PER_W,), jnp.int32),
            pltpu.VMEM((NBUF, BTILE, EMBED_DIM), jnp.float32),
            pltpu.VMEM((NBUF, EMBED_DIM, TPITCH), jnp.float32),
            pltpu.VMEM((SEQ_LEN, EMBED_DIM), jnp.float32),
            pltpu.SemaphoreType.DMA((NBUF,)),
            pltpu.SemaphoreType.DMA((NBUF,)),
        ],
    )
    def k(idx_hbm, table_hbm, pos_hbm, out_hbm, idx_v, rows_v, trans_v,
          pos_v, gsem, osem):
        wid = lax.axis_index("s") * NC + lax.axis_index("c")
        g0 = wid * UNITS_PER_W

        pltpu.sync_copy(
            idx_hbm.at[pl.ds(pl.multiple_of(wid * IDX_PER_W, IDX_PER_W),
                             IDX_PER_W)],
            idx_v)
        pltpu.sync_copy(pos_hbm, pos_v)

        lane = lax.broadcasted_iota(jnp.int32, (L,), 0)

        def fire(uu, p):
            pltpu.async_copy(
                table_hbm.at[idx_v.at[pl.ds(uu * BTILE, BTILE)]],
                rows_v.at[p],
                gsem.at[p],
            )

        def drain_gather(p):
            pltpu.make_async_copy(
                table_hbm.at[pl.ds(0, BTILE)], rows_v.at[p], gsem.at[p]
            ).wait()

        def transform(uu, p):
            s = (g0 + uu) // NJ
            plo = pos_v[s, pl.ds(0, L)]
            phi = pos_v[s, pl.ds(L, L)]

            def rbody(rr, carry):
                for q in range(8):
                    r = rr * 8 + q
                    rcol = jnp.full((L,), 0, jnp.int32) + r
                    lo = rows_v[p, r, pl.ds(0, L)] + plo
                    hi = rows_v[p, r, pl.ds(L, L)] + phi
                    plsc.store_scatter(trans_v.at[p], [lane, rcol], lo)
                    plsc.store_scatter(trans_v.at[p], [lane + L, rcol], hi)
                return carry

            lax.fori_loop(0, BTILE // 8, rbody, 0)

        def fire_stores(uu, p):
            g = g0 + uu
            s = g // NJ
            j = g % NJ
            for i in range(4):
                pltpu.async_copy(
                    trans_v.at[p].at[pl.ds(i * 8, 8), pl.ds(0, BTILE)],
                    out_hbm.at[s * 4 + i, j],
                    osem.at[p],
                )

        def drain_stores(p):
            for i in range(4):
                pltpu.make_async_copy(
                    trans_v.at[p].at[pl.ds(i * 8, 8), pl.ds(0, BTILE)],
                    out_hbm.at[0, 0],
                    osem.at[p],
                ).wait()

        for a in range(AHEAD):
            fire(a, a)

        def quad(h, carry):
            uu = h * NBUF
            for q in range(NBUF):
                u = uu + q
                @pl.when(u + AHEAD < UNITS_PER_W)
                def _(u=u, q=q):
                    @pl.when(u + AHEAD >= NBUF)
                    def _():
                        drain_stores((q + AHEAD) % NBUF)
                    fire(u + AHEAD, (q + AHEAD) % NBUF)
                drain_gather(q)
                transform(u, q)
                fire_stores(u, q)
            return carry

        lax.fori_loop(0, UNITS_PER_W // NBUF, quad, 0)
        for q in range(NBUF):
            drain_stores(q)

    return k


_sc_kernel = _make_kernel()


@jax.jit
def kernel(inputs, token_table, pos_table):
    idx = inputs.astype(jnp.int32).T.reshape(-1)
    out4 = _sc_kernel(idx, token_table, pos_table)
    o5 = out4.reshape(SEQ_LEN, 4, NJ, 8, BTILE)
    return o5.transpose(2, 4, 0, 1, 3).reshape(BATCH, SEQ_LEN, EMBED_DIM)


# speedup vs baseline: 2.1118x; 1.0017x over previous
import functools

import jax
import jax.numpy as jnp
from jax import lax
from jax.experimental import pallas as pl
from jax.experimental.pallas import tpu as pltpu
from jax.experimental.pallas import tpu_sc as plsc

VOCAB = 1000000
SEQ_LEN = 200
EMBED_DIM = 32
BATCH = 4096

NC = 2
NS = 16
NW = NC * NS
L = 16

BTILE = 128
NJ = BATCH // BTILE
UNITS = SEQ_LEN * NJ
UNITS_PER_W = UNITS // NW
IDX_PER_W = UNITS_PER_W * BTILE
TPITCH = BTILE + 1
NBUF = 5
AHEAD = 3


def _make_kernel():
    mesh = plsc.VectorSubcoreMesh(core_axis_name="c", subcore_axis_name="s")

    @functools.partial(
        pl.kernel,
        mesh=mesh,
        out_type=jax.ShapeDtypeStruct((SEQ_LEN * 4, NJ, 8, BTILE),
                                      jnp.float32),
        compiler_params=pltpu.CompilerParams(use_tc_tiling_on_sc=False,
                                             needs_layout_passes=False),
        scratch_types=[
            pltpu.VMEM((IDX_PER_W,), jnp.int32),
            pltpu.VMEM((NBUF, BTILE, EMBED_DIM), jnp.float32),
            pltpu.VMEM((NBUF, EMBED_DIM, TPITCH), jnp.float32),
            pltpu.VMEM((SEQ_LEN, EMBED_DIM), jnp.float32),
            pltpu.SemaphoreType.DMA((NBUF,)),
            pltpu.SemaphoreType.DMA((NBUF,)),
        ],
    )
    def k(idx_hbm, table_hbm, pos_hbm, out_hbm, idx_v, rows_v, trans_v,
          pos_v, gsem, osem):
        wid = lax.axis_index("s") * NC + lax.axis_index("c")
        g0 = wid * UNITS_PER_W

        pltpu.sync_copy(
            idx_hbm.at[pl.ds(pl.multiple_of(wid * IDX_PER_W, IDX_PER_W),
                             IDX_PER_W)],
            idx_v)
        pltpu.sync_copy(pos_hbm, pos_v)

        lane = lax.broadcasted_iota(jnp.int32, (L,), 0)

        def fire(uu, p):
            pltpu.async_copy(
                table_hbm.at[idx_v.at[pl.ds(uu * BTILE, BTILE)]],
                rows_v.at[p],
                gsem.at[p],
            )

        def drain_gather(p):
            pltpu.make_async_copy(
                table_hbm.at[pl.ds(0, BTILE)], rows_v.at[p], gsem.at[p]
            ).wait()

        def transform(uu, p):
            s = (g0 + uu) // NJ
            plo = pos_v[s, pl.ds(0, L)]
            phi = pos_v[s, pl.ds(L, L)]

            def rbody(rr, carry):
                for q in range(8):
                    r = rr * 8 + q
                    rcol = jnp.full((L,), 0, jnp.int32) + r
                    lo = rows_v[p, r, pl.ds(0, L)] + plo
                    hi = rows_v[p, r, pl.ds(L, L)] + phi
                    plsc.store_scatter(trans_v.at[p], [lane, rcol], lo)
                    plsc.store_scatter(trans_v.at[p], [lane + L, rcol], hi)
                return carry

            lax.fori_loop(0, BTILE // 8, rbody, 0)

        def fire_stores(uu, p):
            g = g0 + uu
            s = g // NJ
            j = g % NJ
            for i in range(4):
                pltpu.async_copy(
                    trans_v.at[p].at[pl.ds(i * 8, 8), pl.ds(0, BTILE)],
                    out_hbm.at[s * 4 + i, j],
                    osem.at[p],
                )

        def drain_stores(p):
            for i in range(4):
                pltpu.make_async_copy(
                    trans_v.at[p].at[pl.ds(i * 8, 8), pl.ds(0, BTILE)],
                    out_hbm.at[0, 0],
                    osem.at[p],
                ).wait()

        for a in range(AHEAD):
            fire(a, a)

        def quad(h, carry):
            uu = h * NBUF
            for q in range(NBUF):
                u = uu + q
                @pl.when(u + AHEAD < UNITS_PER_W)
                def _(u=u, q=q):
                    @pl.when(u + AHEAD >= NBUF)
                    def _():
                        drain_stores((q + AHEAD) % NBUF)
                    fire(u + AHEAD, (q + AHEAD) % NBUF)
                drain_gather(q)
                transform(u, q)
                fire_stores(u, q)
            return carry

        lax.fori_loop(0, UNITS_PER_W // NBUF, quad, 0)
        for q in range(NBUF):
            drain_stores(q)

    return k


_sc_kernel = _make_kernel()


@jax.jit
def kernel(inputs, token_table, pos_table):
    idx = inputs.astype(jnp.int32).T.reshape(-1)
    out4 = _sc_kernel(idx, token_table, pos_table)
    o5 = out4.reshape(SEQ_LEN, 4, NJ, 8, BTILE)
    return o5.transpose(2, 4, 0, 1, 3).reshape(BATCH, SEQ_LEN, EMBED_DIM)
